# Initial kernel scaffold; baseline (speedup 1.0000x reference)
import jax
import jax.numpy as jnp
from jax.experimental import pallas as pl


def kernel(x_src, x_dst, seed_time, node_time_src, W_enc_src, b_enc_src, W_enc_dst, b_enc_dst, W_time, b_time, W_self_src_0, W_neigh_ds_0, W_self_dst_0, W_neigh_sd_0, W_self_src_1, W_neigh_ds_1, W_self_dst_1, W_neigh_sd_1, W_head, b_head, edge_index_sd, edge_index_ds, batch_src):
    raise NotImplementedError("write your pallas kernel here")


# speedup vs baseline: 4.7925x; 4.7925x over previous
import functools

import jax
import jax.numpy as jnp
from jax import lax
from jax.experimental import pallas as pl
from jax.experimental.pallas import tpu as pltpu
from jax.experimental.pallas import tpu_sc as plsc

_N = 10000
_E = 320000
_C = 128
_SEEDS = 512
_NF = 16
_NUM_DST = 10000

_NC = 2
_NS = 16
_NW = _NC * _NS

_K = 80
_TILE_EDGES = _E // _NW
_TILE_BATCHES = _TILE_EDGES // _K

_TRASH = _SEEDS
_ACC_SRC_ROWS = 640
_ACC_DST_ROWS = 10240

_BR = 1000

_mesh = functools.partial(
    plsc.VectorSubcoreMesh, core_axis_name="c", subcore_axis_name="s")


def _zero_rows(buf, nrows):
    zv = jnp.zeros((16,), jnp.float32)

    def body(r, carry):
        for k in range(_C // 16):
            buf[r, pl.ds(k * 16, 16)] = zv
        return carry

    lax.fori_loop(0, nrows, body, 0)


_CHUNK_E = 2000
_CHUNK_BATCHES = _CHUNK_E // _K


def _edge_pass(tbl_hbm, i0_hbm, i1_hbm, acc, idx0_v, idx1_v, idx_r, idx_w,
               rows_v, sem, wid, clamp):
    cl = jnp.full((16,), _TRASH if clamp else 0x7FFFFFFF, jnp.int32)

    def chunk_body(ch, carry):
        e0 = wid * _TILE_EDGES + ch * _CHUNK_E
        pltpu.sync_copy(i0_hbm.at[pl.ds(e0, _CHUNK_E)], idx0_v)
        pltpu.sync_copy(i1_hbm.at[pl.ds(e0, _CHUNK_E)], idx1_v)

        def body(j, carry2):
            base = j * _K
            for k in range(_K // 16):
                sl = pl.ds(k * 16, 16)
                idx_r[sl] = idx0_v[pl.ds(base + k * 16, 16)]
                idx_w[sl] = jnp.minimum(idx1_v[pl.ds(base + k * 16, 16)], cl)
            pltpu.async_copy(tbl_hbm.at[idx_r], rows_v, sem).wait()
            pltpu.sync_copy(rows_v, acc.at[idx_w], add=True)
            return carry2

        lax.fori_loop(0, _CHUNK_BATCHES, body, 0)
        return carry

    lax.fori_loop(0, _TILE_EDGES // _CHUNK_E, chunk_body, 0)


@functools.partial(
    pl.kernel,
    mesh=_mesh(),
    out_type=[
        jax.ShapeDtypeStruct((2 * _ACC_DST_ROWS, _C), jnp.float32),
        jax.ShapeDtypeStruct((2 * _SEEDS, _C), jnp.float32),
    ],
    scratch_types=[
        pltpu.VMEM((_CHUNK_E,), jnp.int32),
        pltpu.VMEM((_CHUNK_E,), jnp.int32),
        pltpu.VMEM((_K,), jnp.int32),
        pltpu.VMEM((_K,), jnp.int32),
        pltpu.VMEM((_K, _C), jnp.float32),
        pltpu.VMEM((64, _C), jnp.float32),
        pltpu.VMEM_SHARED((_ACC_DST_ROWS, _C), jnp.float32),
        pltpu.VMEM_SHARED((_ACC_SRC_ROWS, _C), jnp.float32),
        pltpu.SemaphoreType.DMA,
    ],
)
def _segsum0_kernel(hsrc, hdst, sd0, sd1, ds0, ds1, mdst_out, msrc_out,
                    idx0_v, idx1_v, idx_r, idx_w, rows_v, zbuf, acc_dst,
                    acc_src, sem):
    c = lax.axis_index("c")
    s = lax.axis_index("s")
    wid = s * _NC + c
    _zero_rows(zbuf, 64)
    for i in range(10):
        pltpu.sync_copy(zbuf, acc_dst.at[pl.ds(s * 640 + i * 64, 64)])
    pltpu.sync_copy(zbuf.at[pl.ds(0, 40)], acc_src.at[pl.ds(s * 40, 40)])
    plsc.subcore_barrier()
    _edge_pass(hsrc, sd0, sd1, acc_dst, idx0_v, idx1_v, idx_r, idx_w,
               rows_v, sem, wid, False)
    _edge_pass(hdst, ds0, ds1, acc_src, idx0_v, idx1_v, idx_r, idx_w,
               rows_v, sem, wid, True)
    plsc.subcore_barrier()
    for i in range(10):
        r0 = s * 640 + i * 64
        pltpu.sync_copy(acc_dst.at[pl.ds(r0, 64)], zbuf)
        pltpu.sync_copy(zbuf, mdst_out.at[pl.ds(c * _ACC_DST_ROWS + r0, 64)])
    pltpu.sync_copy(acc_src.at[pl.ds(s * 32, 32)], zbuf.at[pl.ds(0, 32)])
    pltpu.sync_copy(zbuf.at[pl.ds(0, 32)],
                    msrc_out.at[pl.ds(c * _SEEDS + s * 32, 32)])


@functools.partial(
    pl.kernel,
    mesh=_mesh(),
    out_type=jax.ShapeDtypeStruct((2 * _SEEDS, _C), jnp.float32),
    scratch_types=[
        pltpu.VMEM((_CHUNK_E,), jnp.int32),
        pltpu.VMEM((_CHUNK_E,), jnp.int32),
        pltpu.VMEM((_K,), jnp.int32),
        pltpu.VMEM((_K,), jnp.int32),
        pltpu.VMEM((_K, _C), jnp.float32),
        pltpu.VMEM((40, _C), jnp.float32),
        pltpu.VMEM_SHARED((_ACC_SRC_ROWS, _C), jnp.float32),
        pltpu.SemaphoreType.DMA,
    ],
)
def _segsum1_kernel(hdst2, ds0, ds1, msrc2_out,
                    idx0_v, idx1_v, idx_r, idx_w, rows_v, zbuf, acc_src, sem):
    c = lax.axis_index("c")
    s = lax.axis_index("s")
    wid = s * _NC + c
    _zero_rows(zbuf, 40)
    pltpu.sync_copy(zbuf, acc_src.at[pl.ds(s * 40, 40)])
    plsc.subcore_barrier()
    _edge_pass(hdst2, ds0, ds1, acc_src, idx0_v, idx1_v, idx_r, idx_w,
               rows_v, sem, wid, True)
    plsc.subcore_barrier()
    pltpu.sync_copy(acc_src.at[pl.ds(s * 32, 32)], zbuf.at[pl.ds(0, 32)])
    pltpu.sync_copy(zbuf.at[pl.ds(0, 32)],
                    msrc2_out.at[pl.ds(c * _SEEDS + s * 32, 32)])


def _enc_body(xs_ref, xd_ref, batch_ref, nt_ref, st_ref, fr_ref, Wes_ref,
              bes_ref, Wed_ref, bed_ref, Wt_ref, bt_ref, hs_ref, hd_ref):
    f32 = jnp.float32
    hs = jnp.dot(xs_ref[...], Wes_ref[...], preferred_element_type=f32)
    hs = hs + bes_ref[...]
    batch = batch_ref[...][0]
    oh = batch == lax.broadcasted_iota(jnp.int32, (_BR, _SEEDS), 1)
    seedg = jnp.sum(jnp.where(oh, st_ref[...], 0.0), axis=1, keepdims=True)
    rel = seedg - nt_ref[...][0]
    ang = rel * fr_ref[...]
    pe = jnp.concatenate([jnp.sin(ang), jnp.cos(ang)], axis=1)
    hs = hs + jnp.dot(pe, Wt_ref[...], preferred_element_type=f32)
    hs_ref[...] = hs + bt_ref[...]
    hd = jnp.dot(xd_ref[...], Wed_ref[...], preferred_element_type=f32)
    hd_ref[...] = hd + bed_ref[...]


def _encoder(x_src, x_dst, batch_col, nt_col, seed_row, freqs,
             Wes, bes, Wed, bed, Wt, bt):
    nb = _N // _BR
    row = pl.BlockSpec((_BR, _C), lambda i: (i, 0))
    col = pl.BlockSpec((1, _BR, 1), lambda i: (i, 0, 0))
    return pl.pallas_call(
        _enc_body,
        grid=(nb,),
        in_specs=[
            row,
            row,
            col,
            col,
            pl.BlockSpec((1, _SEEDS), lambda i: (0, 0)),
            pl.BlockSpec((1, _NF), lambda i: (0, 0)),
            pl.BlockSpec((_C, _C), lambda i: (0, 0)),
            pl.BlockSpec((1, _C), lambda i: (0, 0)),
            pl.BlockSpec((_C, _C), lambda i: (0, 0)),
            pl.BlockSpec((1, _C), lambda i: (0, 0)),
            pl.BlockSpec((2 * _NF, _C), lambda i: (0, 0)),
            pl.BlockSpec((1, _C), lambda i: (0, 0)),
        ],
        out_specs=[row, row],
        out_shape=[
            jax.ShapeDtypeStruct((_N, _C), jnp.float32),
            jax.ShapeDtypeStruct((_N, _C), jnp.float32),
        ],
    )(x_src, x_dst, batch_col, nt_col, seed_row, freqs,
      Wes, bes, Wed, bed, Wt, bt)


def _l0_body(hd_ref, m0_ref, m1_ref, Wsd_ref, Wn_ref, out_ref):
    f32 = jnp.float32
    m = m0_ref[...] + m1_ref[...]
    x = jnp.dot(hd_ref[...], Wsd_ref[...], preferred_element_type=f32)
    x = x + jnp.dot(m, Wn_ref[...], preferred_element_type=f32)
    out_ref[...] = jnp.maximum(x, 0.0)


def _layer0_dst(hd, p0, p1, Wsd, Wn):
    nb = _N // _BR
    row = pl.BlockSpec((_BR, _C), lambda i: (i, 0))
    sq = pl.BlockSpec((_C, _C), lambda i: (0, 0))
    return pl.pallas_call(
        _l0_body,
        grid=(nb,),
        in_specs=[row, row, row, sq, sq],
        out_specs=row,
        out_shape=jax.ShapeDtypeStruct((_N, _C), jnp.float32),
    )(hd, p0, p1, Wsd, Wn)


def _head_body(hs_ref, m0_ref, m1_ref, n0_ref, n1_ref, Wss0_ref, Wnd0_ref,
               Wss1_ref, Wnd1_ref, Wh_ref, bh_ref, out_ref):
    f32 = jnp.float32
    m = m0_ref[...] + m1_ref[...]
    h2 = jnp.dot(hs_ref[...], Wss0_ref[...], preferred_element_type=f32)
    h2 = jnp.maximum(h2 + jnp.dot(m, Wnd0_ref[...],
                                  preferred_element_type=f32), 0.0)
    n = n0_ref[...] + n1_ref[...]
    h3 = jnp.dot(h2, Wss1_ref[...], preferred_element_type=f32)
    h3 = h3 + jnp.dot(n, Wnd1_ref[...], preferred_element_type=f32)
    out_ref[...] = jnp.dot(h3, Wh_ref[...],
                           preferred_element_type=f32) + bh_ref[...]


def _head(hs, msrc_flat, msrc2_flat, Wss0, Wnd0, Wss1, Wnd1, Wh, bh):
    top = pl.BlockSpec((_SEEDS, _C), lambda i: (0, 0))
    bot = pl.BlockSpec((_SEEDS, _C), lambda i: (1, 0))
    sq = pl.BlockSpec((_C, _C), lambda i: (0, 0))
    return pl.pallas_call(
        _head_body,
        grid=(1,),
        in_specs=[
            top, top, bot, top, bot, sq, sq, sq, sq,
            pl.BlockSpec((_C, _NUM_DST), lambda i: (0, 0)),
            pl.BlockSpec((1, _NUM_DST), lambda i: (0, 0)),
        ],
        out_specs=pl.BlockSpec((_SEEDS, _NUM_DST), lambda i: (0, 0)),
        out_shape=jax.ShapeDtypeStruct((_SEEDS, _NUM_DST), jnp.float32),
    )(hs, msrc_flat, msrc_flat, msrc2_flat, msrc2_flat,
      Wss0, Wnd0, Wss1, Wnd1, Wh, bh)


def kernel(x_src, x_dst, seed_time, node_time_src, W_enc_src, b_enc_src,
           W_enc_dst, b_enc_dst, W_time, b_time, W_self_src_0, W_neigh_ds_0,
           W_self_dst_0, W_neigh_sd_0, W_self_src_1, W_neigh_ds_1,
           W_self_dst_1, W_neigh_sd_1, W_head, b_head, edge_index_sd,
           edge_index_ds, batch_src):
    f32 = jnp.float32
    sd = edge_index_sd.astype(jnp.int32)
    ds = edge_index_ds.astype(jnp.int32)
    sd0, sd1 = sd[0], sd[1]
    ds0, ds1 = ds[0], ds[1]
    batch_col = batch_src.astype(jnp.int32).reshape(_N // _BR, _BR, 1)
    nt_col = node_time_src.reshape(_N // _BR, _BR, 1)
    seed_row = seed_time.reshape(1, _SEEDS)
    freqs = (2.0 ** jnp.arange(_NF, dtype=f32)).reshape(1, _NF)

    hs, hd = _encoder(x_src, x_dst, batch_col, nt_col, seed_row, freqs,
                      W_enc_src, b_enc_src.reshape(1, _C),
                      W_enc_dst, b_enc_dst.reshape(1, _C),
                      W_time, b_time.reshape(1, _C))

    mdst_flat, msrc_flat = _segsum0_kernel(hs, hd, sd0, sd1, ds0, ds1)
    p0 = mdst_flat[:_N]
    p1 = mdst_flat[_ACC_DST_ROWS:_ACC_DST_ROWS + _N]
    hd2 = _layer0_dst(hd, p0, p1, W_self_dst_0, W_neigh_sd_0)
    msrc2_flat = _segsum1_kernel(hd2, ds0, ds1)
    out = _head(hs, msrc_flat, msrc2_flat, W_self_src_0, W_neigh_ds_0,
                W_self_src_1, W_neigh_ds_1, W_head,
                b_head.reshape(1, _NUM_DST))
    return out


# speedup vs baseline: 5.6901x; 1.1873x over previous
import functools

import jax
import jax.numpy as jnp
from jax import lax
from jax.experimental import pallas as pl
from jax.experimental.pallas import tpu as pltpu
from jax.experimental.pallas import tpu_sc as plsc

_N = 10000
_E = 320000
_C = 128
_SEEDS = 512
_NF = 16
_NUM_DST = 10000

_NC = 2
_NS = 16
_NW = _NC * _NS

_K = 80
_TILE_EDGES = _E // _NW
_TILE_BATCHES = _TILE_EDGES // _K

_TRASH = _SEEDS
_ACC_SRC_ROWS = 640
_ACC_DST_ROWS = 10240

_BR = 1000

_mesh = functools.partial(
    plsc.VectorSubcoreMesh, core_axis_name="c", subcore_axis_name="s")


def _zero_rows(buf, nrows):
    zv = jnp.zeros((16,), jnp.float32)

    def body(r, carry):
        for k in range(_C // 16):
            buf[r, pl.ds(k * 16, 16)] = zv
        return carry

    lax.fori_loop(0, nrows, body, 0)


_CHUNK_E = 2000
_CHUNK_BATCHES = _CHUNK_E // _K
_COMP_WORDS = _CHUNK_E + _K


def _stage_batch(src_v, dst_ref, base):
    for k in range(_K // 16):
        dst_ref[pl.ds(k * 16, 16)] = src_v[pl.ds(base + k * 16, 16)]


def _db_pass(tbl_hbm, i0_hbm, i1_hbm, acc, idx0_v, idx1_v, ir0, iw0, ir1,
             iw1, rows0, rows1, sem0, sem1, wid, clamp):
    cl = jnp.full((16,), clamp, jnp.int32)

    def stage(b, ir, iw):
        for k in range(_K // 16):
            sl = pl.ds(k * 16, 16)
            ir[sl] = idx0_v[pl.ds(b * _K + k * 16, 16)]
            iw[sl] = jnp.minimum(idx1_v[pl.ds(b * _K + k * 16, 16)], cl)

    def chunk_body(ch, carry):
        e0 = wid * _TILE_EDGES + ch * _CHUNK_E
        pltpu.sync_copy(i0_hbm.at[pl.ds(e0, _CHUNK_E)], idx0_v)
        pltpu.sync_copy(i1_hbm.at[pl.ds(e0, _CHUNK_E)], idx1_v)
        stage(0, ir0, iw0)
        cp = pltpu.async_copy(tbl_hbm.at[ir0], rows0, sem0)
        prev = (cp, iw0, rows0)
        for b in range(1, _CHUNK_BATCHES):
            if b % 2:
                ir, iw, rows, sem = ir1, iw1, rows1, sem1
            else:
                ir, iw, rows, sem = ir0, iw0, rows0, sem0
            stage(b, ir, iw)
            cp = pltpu.async_copy(tbl_hbm.at[ir], rows, sem)
            pcp, piw, prows = prev
            pcp.wait()
            pltpu.sync_copy(prows, acc.at[piw], add=True)
            prev = (cp, iw, rows)
        pcp, piw, prows = prev
        pcp.wait()
        pltpu.sync_copy(prows, acc.at[piw], add=True)
        return carry

    lax.fori_loop(0, _TILE_EDGES // _CHUNK_E, chunk_body, 0)


@functools.partial(
    pl.kernel,
    mesh=_mesh(),
    out_type=[
        jax.ShapeDtypeStruct((2 * _ACC_DST_ROWS, _C), jnp.float32),
        jax.ShapeDtypeStruct((2 * _SEEDS, _C), jnp.float32),
    ],
    scratch_types=[
        pltpu.VMEM((_CHUNK_E,), jnp.int32),
        pltpu.VMEM((_CHUNK_E,), jnp.int32),
        pltpu.VMEM((_K,), jnp.int32),
        pltpu.VMEM((_K,), jnp.int32),
        pltpu.VMEM((_K,), jnp.int32),
        pltpu.VMEM((_K,), jnp.int32),
        pltpu.VMEM((_K, _C), jnp.float32),
        pltpu.VMEM((_K, _C), jnp.float32),
        pltpu.VMEM((32, _C), jnp.float32),
        pltpu.VMEM_SHARED((_ACC_DST_ROWS, _C), jnp.float32),
        pltpu.VMEM_SHARED((_ACC_SRC_ROWS, _C), jnp.float32),
        pltpu.SemaphoreType.DMA,
        pltpu.SemaphoreType.DMA,
    ],
)
def _segsum0_kernel(hsrc, hdst, sd0, sd1, ds0, ds1, mdst_out, msrc_out,
                    idx0_v, idx1_v, ir0, iw0, ir1, iw1,
                    rows0, rows1, zbuf, acc_dst, acc_src, sem0, sem1):
    c = lax.axis_index("c")
    s = lax.axis_index("s")
    wid = s * _NC + c
    _zero_rows(zbuf, 32)
    for i in range(20):
        pltpu.sync_copy(zbuf, acc_dst.at[pl.ds(s * 640 + i * 32, 32)])
    pltpu.sync_copy(zbuf, acc_src.at[pl.ds(s * 40, 32)])
    pltpu.sync_copy(zbuf.at[pl.ds(0, 8)], acc_src.at[pl.ds(s * 40 + 32, 8)])
    plsc.subcore_barrier()
    _db_pass(hsrc, sd0, sd1, acc_dst, idx0_v, idx1_v, ir0, iw0, ir1, iw1,
             rows0, rows1, sem0, sem1, wid, _ACC_DST_ROWS - 1)
    _db_pass(hdst, ds0, ds1, acc_src, idx0_v, idx1_v, ir0, iw0, ir1, iw1,
             rows0, rows1, sem0, sem1, wid, _TRASH)
    plsc.subcore_barrier()
    for i in range(20):
        r0 = s * 640 + i * 32
        pltpu.sync_copy(acc_dst.at[pl.ds(r0, 32)], zbuf)
        pltpu.sync_copy(zbuf, mdst_out.at[pl.ds(c * _ACC_DST_ROWS + r0, 32)])
    pltpu.sync_copy(acc_src.at[pl.ds(s * 32, 32)], zbuf)
    pltpu.sync_copy(zbuf, msrc_out.at[pl.ds(c * _SEEDS + s * 32, 32)])


@functools.partial(
    pl.kernel,
    mesh=_mesh(),
    out_type=jax.ShapeDtypeStruct((2 * _SEEDS, _C), jnp.float32),
    scratch_types=[
        pltpu.VMEM((_CHUNK_E,), jnp.int32),
        pltpu.VMEM((_CHUNK_E,), jnp.int32),
        pltpu.VMEM((_K,), jnp.int32),
        pltpu.VMEM((_K,), jnp.int32),
        pltpu.VMEM((_K,), jnp.int32),
        pltpu.VMEM((_K,), jnp.int32),
        pltpu.VMEM((_K, _C), jnp.float32),
        pltpu.VMEM((_K, _C), jnp.float32),
        pltpu.VMEM((40, _C), jnp.float32),
        pltpu.VMEM_SHARED((_ACC_SRC_ROWS, _C), jnp.float32),
        pltpu.SemaphoreType.DMA,
        pltpu.SemaphoreType.DMA,
    ],
)
def _segsum1_kernel(hdst2, ds0, ds1, msrc2_out, idx0_v, idx1_v, ir0, iw0,
                    ir1, iw1, rows0, rows1, zbuf, acc_src, sem0, sem1):
    c = lax.axis_index("c")
    s = lax.axis_index("s")
    wid = s * _NC + c
    _zero_rows(zbuf, 40)
    pltpu.sync_copy(zbuf, acc_src.at[pl.ds(s * 40, 40)])
    plsc.subcore_barrier()
    _db_pass(hdst2, ds0, ds1, acc_src, idx0_v, idx1_v, ir0, iw0, ir1, iw1,
             rows0, rows1, sem0, sem1, wid, _TRASH)
    plsc.subcore_barrier()
    pltpu.sync_copy(acc_src.at[pl.ds(s * 32, 32)], zbuf.at[pl.ds(0, 32)])
    pltpu.sync_copy(zbuf.at[pl.ds(0, 32)],
                    msrc2_out.at[pl.ds(c * _SEEDS + s * 32, 32)])


def _enc_body(xs_ref, xd_ref, batch_ref, nt_ref, st_ref, fr_ref, Wes_ref,
              bes_ref, Wed_ref, bed_ref, Wt_ref, bt_ref, hs_ref, hd_ref):
    f32 = jnp.float32
    hs = jnp.dot(xs_ref[...], Wes_ref[...], preferred_element_type=f32)
    hs = hs + bes_ref[...]
    batch = batch_ref[...][0]
    oh = batch == lax.broadcasted_iota(jnp.int32, (_BR, _SEEDS), 1)
    seedg = jnp.sum(jnp.where(oh, st_ref[...], 0.0), axis=1, keepdims=True)
    rel = seedg - nt_ref[...][0]
    ang = rel * fr_ref[...]
    pe = jnp.concatenate([jnp.sin(ang), jnp.cos(ang)], axis=1)
    hs = hs + jnp.dot(pe, Wt_ref[...], preferred_element_type=f32)
    hs_ref[...] = hs + bt_ref[...]
    hd = jnp.dot(xd_ref[...], Wed_ref[...], preferred_element_type=f32)
    hd_ref[...] = hd + bed_ref[...]


def _encoder(x_src, x_dst, batch_col, nt_col, seed_row, freqs,
             Wes, bes, Wed, bed, Wt, bt):
    nb = _N // _BR
    row = pl.BlockSpec((_BR, _C), lambda i: (i, 0))
    col = pl.BlockSpec((1, _BR, 1), lambda i: (i, 0, 0))
    return pl.pallas_call(
        _enc_body,
        grid=(nb,),
        in_specs=[
            row,
            row,
            col,
            col,
            pl.BlockSpec((1, _SEEDS), lambda i: (0, 0)),
            pl.BlockSpec((1, _NF), lambda i: (0, 0)),
            pl.BlockSpec((_C, _C), lambda i: (0, 0)),
            pl.BlockSpec((1, _C), lambda i: (0, 0)),
            pl.BlockSpec((_C, _C), lambda i: (0, 0)),
            pl.BlockSpec((1, _C), lambda i: (0, 0)),
            pl.BlockSpec((2 * _NF, _C), lambda i: (0, 0)),
            pl.BlockSpec((1, _C), lambda i: (0, 0)),
        ],
        out_specs=[row, row],
        out_shape=[
            jax.ShapeDtypeStruct((_N, _C), jnp.float32),
            jax.ShapeDtypeStruct((_N, _C), jnp.float32),
        ],
    )(x_src, x_dst, batch_col, nt_col, seed_row, freqs,
      Wes, bes, Wed, bed, Wt, bt)


def _l0_body(hd_ref, m0_ref, m1_ref, Wsd_ref, Wn_ref, out_ref):
    f32 = jnp.float32
    m = m0_ref[...] + m1_ref[...]
    x = jnp.dot(hd_ref[...], Wsd_ref[...], preferred_element_type=f32)
    x = x + jnp.dot(m, Wn_ref[...], preferred_element_type=f32)
    out_ref[...] = jnp.maximum(x, 0.0)


def _layer0_dst(hd, p0, p1, Wsd, Wn):
    nb = _N // _BR
    row = pl.BlockSpec((_BR, _C), lambda i: (i, 0))
    sq = pl.BlockSpec((_C, _C), lambda i: (0, 0))
    return pl.pallas_call(
        _l0_body,
        grid=(nb,),
        in_specs=[row, row, row, sq, sq],
        out_specs=row,
        out_shape=jax.ShapeDtypeStruct((_N, _C), jnp.float32),
    )(hd, p0, p1, Wsd, Wn)


def _head_body(hs_ref, m0_ref, m1_ref, n0_ref, n1_ref, Wss0_ref, Wnd0_ref,
               Wss1_ref, Wnd1_ref, Wh_ref, bh_ref, out_ref):
    f32 = jnp.float32
    m = m0_ref[...] + m1_ref[...]
    h2 = jnp.dot(hs_ref[...], Wss0_ref[...], preferred_element_type=f32)
    h2 = jnp.maximum(h2 + jnp.dot(m, Wnd0_ref[...],
                                  preferred_element_type=f32), 0.0)
    n = n0_ref[...] + n1_ref[...]
    h3 = jnp.dot(h2, Wss1_ref[...], preferred_element_type=f32)
    h3 = h3 + jnp.dot(n, Wnd1_ref[...], preferred_element_type=f32)
    out_ref[...] = jnp.dot(h3, Wh_ref[...],
                           preferred_element_type=f32) + bh_ref[...]


def _head(hs, msrc_flat, msrc2_flat, Wss0, Wnd0, Wss1, Wnd1, Wh, bh):
    top = pl.BlockSpec((_SEEDS, _C), lambda i: (0, 0))
    bot = pl.BlockSpec((_SEEDS, _C), lambda i: (1, 0))
    sq = pl.BlockSpec((_C, _C), lambda i: (0, 0))
    return pl.pallas_call(
        _head_body,
        grid=(1,),
        in_specs=[
            top, top, bot, top, bot, sq, sq, sq, sq,
            pl.BlockSpec((_C, _NUM_DST), lambda i: (0, 0)),
            pl.BlockSpec((1, _NUM_DST), lambda i: (0, 0)),
        ],
        out_specs=pl.BlockSpec((_SEEDS, _NUM_DST), lambda i: (0, 0)),
        out_shape=jax.ShapeDtypeStruct((_SEEDS, _NUM_DST), jnp.float32),
    )(hs, msrc_flat, msrc_flat, msrc2_flat, msrc2_flat,
      Wss0, Wnd0, Wss1, Wnd1, Wh, bh)


def kernel(x_src, x_dst, seed_time, node_time_src, W_enc_src, b_enc_src,
           W_enc_dst, b_enc_dst, W_time, b_time, W_self_src_0, W_neigh_ds_0,
           W_self_dst_0, W_neigh_sd_0, W_self_src_1, W_neigh_ds_1,
           W_self_dst_1, W_neigh_sd_1, W_head, b_head, edge_index_sd,
           edge_index_ds, batch_src):
    f32 = jnp.float32
    sd = edge_index_sd.astype(jnp.int32)
    ds = edge_index_ds.astype(jnp.int32)
    sd0, sd1 = sd[0], sd[1]
    ds0, ds1 = ds[0], ds[1]
    batch_col = batch_src.astype(jnp.int32).reshape(_N // _BR, _BR, 1)
    nt_col = node_time_src.reshape(_N // _BR, _BR, 1)
    seed_row = seed_time.reshape(1, _SEEDS)
    freqs = (2.0 ** jnp.arange(_NF, dtype=f32)).reshape(1, _NF)

    hs, hd = _encoder(x_src, x_dst, batch_col, nt_col, seed_row, freqs,
                      W_enc_src, b_enc_src.reshape(1, _C),
                      W_enc_dst, b_enc_dst.reshape(1, _C),
                      W_time, b_time.reshape(1, _C))

    mdst_flat, msrc_flat = _segsum0_kernel(hs, hd, sd0, sd1, ds0, ds1)
    p0 = mdst_flat[:_N]
    p1 = mdst_flat[_ACC_DST_ROWS:_ACC_DST_ROWS + _N]
    hd2 = _layer0_dst(hd, p0, p1, W_self_dst_0, W_neigh_sd_0)
    msrc2_flat = _segsum1_kernel(hd2, ds0, ds1)
    out = _head(hs, msrc_flat, msrc2_flat, W_self_src_0, W_neigh_ds_0,
                W_self_src_1, W_neigh_ds_1, W_head,
                b_head.reshape(1, _NUM_DST))
    return out


# speedup vs baseline: 7.3862x; 1.2981x over previous
import functools

import jax
import jax.numpy as jnp
from jax import lax
from jax.experimental import pallas as pl
from jax.experimental.pallas import tpu as pltpu
from jax.experimental.pallas import tpu_sc as plsc

_N = 10000
_E = 320000
_C = 128
_SEEDS = 512
_NF = 16
_NUM_DST = 10000

_NC = 2
_NS = 16
_NW = _NC * _NS

_K = 80
_TILE_EDGES = _E // _NW
_TILE_BATCHES = _TILE_EDGES // _K

_TRASH = _SEEDS
_ACC_SRC_ROWS = 640
_ACC_DST_ROWS = 10240

_BR = 1000

_mesh = functools.partial(
    plsc.VectorSubcoreMesh, core_axis_name="c", subcore_axis_name="s")


def _zero_rows(buf, nrows):
    zv = jnp.zeros((16,), jnp.float32)

    def body(r, carry):
        for k in range(_C // 16):
            buf[r, pl.ds(k * 16, 16)] = zv
        return carry

    lax.fori_loop(0, nrows, body, 0)


_CHUNK_E = 2000
_CHUNK_BATCHES = _CHUNK_E // _K
_COMP_WORDS = _CHUNK_E + _K


def _stage_batch(src_v, dst_ref, base):
    for k in range(_K // 16):
        dst_ref[pl.ds(k * 16, 16)] = src_v[pl.ds(base + k * 16, 16)]


def _db_pass(tbl_hbm, i0_hbm, i1_hbm, acc, idx0_v, idx1_v, ir0, iw0, ir1,
             iw1, rows0, rows1, sem0, sem1, wid, clamp):
    if clamp is not None:
        cl = jnp.full((16,), clamp, jnp.int32)
        lane = lax.iota(jnp.int32, 16)

    def stage(b, ir, iw):
        for k in range(_K // 16):
            sl = pl.ds(k * 16, 16)
            ir[sl] = idx0_v[pl.ds(b * _K + k * 16, 16)]
            seg = idx1_v[pl.ds(b * _K + k * 16, 16)]
            if clamp is None:
                iw[sl] = seg
            else:
                iw[sl] = jnp.where(seg < cl, seg, clamp + k * 16 + lane)

    def chunk_body(ch, carry):
        e0 = wid * _TILE_EDGES + ch * _CHUNK_E
        pltpu.sync_copy(i0_hbm.at[pl.ds(e0, _CHUNK_E)], idx0_v)
        pltpu.sync_copy(i1_hbm.at[pl.ds(e0, _CHUNK_E)], idx1_v)
        stage(0, ir0, iw0)
        cp = pltpu.async_copy(tbl_hbm.at[ir0], rows0, sem0)
        prev = (cp, iw0, rows0)
        for b in range(1, _CHUNK_BATCHES):
            if b % 2:
                ir, iw, rows, sem = ir1, iw1, rows1, sem1
            else:
                ir, iw, rows, sem = ir0, iw0, rows0, sem0
            stage(b, ir, iw)
            cp = pltpu.async_copy(tbl_hbm.at[ir], rows, sem)
            pcp, piw, prows = prev
            pcp.wait()
            pltpu.sync_copy(prows, acc.at[piw], add=True)
            prev = (cp, iw, rows)
        pcp, piw, prows = prev
        pcp.wait()
        pltpu.sync_copy(prows, acc.at[piw], add=True)
        return carry

    lax.fori_loop(0, _TILE_EDGES // _CHUNK_E, chunk_body, 0)


@functools.partial(
    pl.kernel,
    mesh=_mesh(),
    out_type=[
        jax.ShapeDtypeStruct((2 * _ACC_DST_ROWS, _C), jnp.float32),
        jax.ShapeDtypeStruct((2 * _SEEDS, _C), jnp.float32),
    ],
    scratch_types=[
        pltpu.VMEM((_CHUNK_E,), jnp.int32),
        pltpu.VMEM((_CHUNK_E,), jnp.int32),
        pltpu.VMEM((_K,), jnp.int32),
        pltpu.VMEM((_K,), jnp.int32),
        pltpu.VMEM((_K,), jnp.int32),
        pltpu.VMEM((_K,), jnp.int32),
        pltpu.VMEM((_K, _C), jnp.float32),
        pltpu.VMEM((_K, _C), jnp.float32),
        pltpu.VMEM((32, _C), jnp.float32),
        pltpu.VMEM_SHARED((_ACC_DST_ROWS, _C), jnp.float32),
        pltpu.VMEM_SHARED((_ACC_SRC_ROWS, _C), jnp.float32),
        pltpu.SemaphoreType.DMA,
        pltpu.SemaphoreType.DMA,
    ],
)
def _segsum0_kernel(hsrc, hdst, sd0, sd1, ds0, ds1, mdst_out, msrc_out,
                    idx0_v, idx1_v, ir0, iw0, ir1, iw1,
                    rows0, rows1, zbuf, acc_dst, acc_src, sem0, sem1):
    c = lax.axis_index("c")
    s = lax.axis_index("s")
    wid = s * _NC + c
    _zero_rows(zbuf, 32)
    for i in range(20):
        pltpu.sync_copy(zbuf, acc_dst.at[pl.ds(s * 640 + i * 32, 32)])
    pltpu.sync_copy(zbuf, acc_src.at[pl.ds(s * 40, 32)])
    pltpu.sync_copy(zbuf.at[pl.ds(0, 8)], acc_src.at[pl.ds(s * 40 + 32, 8)])
    plsc.subcore_barrier()
    _db_pass(hsrc, sd0, sd1, acc_dst, idx0_v, idx1_v, ir0, iw0, ir1, iw1,
             rows0, rows1, sem0, sem1, wid, None)
    _db_pass(hdst, ds0, ds1, acc_src, idx0_v, idx1_v, ir0, iw0, ir1, iw1,
             rows0, rows1, sem0, sem1, wid, _TRASH)
    plsc.subcore_barrier()
    for i in range(20):
        r0 = s * 640 + i * 32
        pltpu.sync_copy(acc_dst.at[pl.ds(r0, 32)], zbuf)
        pltpu.sync_copy(zbuf, mdst_out.at[pl.ds(c * _ACC_DST_ROWS + r0, 32)])
    pltpu.sync_copy(acc_src.at[pl.ds(s * 32, 32)], zbuf)
    pltpu.sync_copy(zbuf, msrc_out.at[pl.ds(c * _SEEDS + s * 32, 32)])


@functools.partial(
    pl.kernel,
    mesh=_mesh(),
    out_type=jax.ShapeDtypeStruct((2 * _SEEDS, _C), jnp.float32),
    scratch_types=[
        pltpu.VMEM((_CHUNK_E,), jnp.int32),
        pltpu.VMEM((_CHUNK_E,), jnp.int32),
        pltpu.VMEM((_K,), jnp.int32),
        pltpu.VMEM((_K,), jnp.int32),
        pltpu.VMEM((_K,), jnp.int32),
        pltpu.VMEM((_K,), jnp.int32),
        pltpu.VMEM((_K, _C), jnp.float32),
        pltpu.VMEM((_K, _C), jnp.float32),
        pltpu.VMEM((40, _C), jnp.float32),
        pltpu.VMEM_SHARED((_ACC_SRC_ROWS, _C), jnp.float32),
        pltpu.SemaphoreType.DMA,
        pltpu.SemaphoreType.DMA,
    ],
)
def _segsum1_kernel(hdst2, ds0, ds1, msrc2_out, idx0_v, idx1_v, ir0, iw0,
                    ir1, iw1, rows0, rows1, zbuf, acc_src, sem0, sem1):
    c = lax.axis_index("c")
    s = lax.axis_index("s")
    wid = s * _NC + c
    _zero_rows(zbuf, 40)
    pltpu.sync_copy(zbuf, acc_src.at[pl.ds(s * 40, 40)])
    plsc.subcore_barrier()
    _db_pass(hdst2, ds0, ds1, acc_src, idx0_v, idx1_v, ir0, iw0, ir1, iw1,
             rows0, rows1, sem0, sem1, wid, _TRASH)
    plsc.subcore_barrier()
    pltpu.sync_copy(acc_src.at[pl.ds(s * 32, 32)], zbuf.at[pl.ds(0, 32)])
    pltpu.sync_copy(zbuf.at[pl.ds(0, 32)],
                    msrc2_out.at[pl.ds(c * _SEEDS + s * 32, 32)])


def _enc_body(xs_ref, xd_ref, batch_ref, nt_ref, st_ref, fr_ref, Wes_ref,
              bes_ref, Wed_ref, bed_ref, Wt_ref, bt_ref, hs_ref, hd_ref):
    f32 = jnp.float32
    hs = jnp.dot(xs_ref[...], Wes_ref[...], preferred_element_type=f32)
    hs = hs + bes_ref[...]
    batch = batch_ref[...][0]
    oh = batch == lax.broadcasted_iota(jnp.int32, (_BR, _SEEDS), 1)
    seedg = jnp.sum(jnp.where(oh, st_ref[...], 0.0), axis=1, keepdims=True)
    rel = seedg - nt_ref[...][0]
    ang = rel * fr_ref[...]
    pe = jnp.concatenate([jnp.sin(ang), jnp.cos(ang)], axis=1)
    hs = hs + jnp.dot(pe, Wt_ref[...], preferred_element_type=f32)
    hs_ref[...] = hs + bt_ref[...]
    hd = jnp.dot(xd_ref[...], Wed_ref[...], preferred_element_type=f32)
    hd_ref[...] = hd + bed_ref[...]


def _encoder(x_src, x_dst, batch_col, nt_col, seed_row, freqs,
             Wes, bes, Wed, bed, Wt, bt):
    nb = _N // _BR
    row = pl.BlockSpec((_BR, _C), lambda i: (i, 0))
    col = pl.BlockSpec((1, _BR, 1), lambda i: (i, 0, 0))
    return pl.pallas_call(
        _enc_body,
        grid=(nb,),
        in_specs=[
            row,
            row,
            col,
            col,
            pl.BlockSpec((1, _SEEDS), lambda i: (0, 0)),
            pl.BlockSpec((1, _NF), lambda i: (0, 0)),
            pl.BlockSpec((_C, _C), lambda i: (0, 0)),
            pl.BlockSpec((1, _C), lambda i: (0, 0)),
            pl.BlockSpec((_C, _C), lambda i: (0, 0)),
            pl.BlockSpec((1, _C), lambda i: (0, 0)),
            pl.BlockSpec((2 * _NF, _C), lambda i: (0, 0)),
            pl.BlockSpec((1, _C), lambda i: (0, 0)),
        ],
        out_specs=[row, row],
        out_shape=[
            jax.ShapeDtypeStruct((_N, _C), jnp.float32),
            jax.ShapeDtypeStruct((_N, _C), jnp.float32),
        ],
    )(x_src, x_dst, batch_col, nt_col, seed_row, freqs,
      Wes, bes, Wed, bed, Wt, bt)


def _l0_body(hd_ref, m0_ref, m1_ref, Wsd_ref, Wn_ref, out_ref):
    f32 = jnp.float32
    m = m0_ref[...] + m1_ref[...]
    x = jnp.dot(hd_ref[...], Wsd_ref[...], preferred_element_type=f32)
    x = x + jnp.dot(m, Wn_ref[...], preferred_element_type=f32)
    out_ref[...] = jnp.maximum(x, 0.0)


def _layer0_dst(hd, p0, p1, Wsd, Wn):
    nb = _N // _BR
    row = pl.BlockSpec((_BR, _C), lambda i: (i, 0))
    sq = pl.BlockSpec((_C, _C), lambda i: (0, 0))
    return pl.pallas_call(
        _l0_body,
        grid=(nb,),
        in_specs=[row, row, row, sq, sq],
        out_specs=row,
        out_shape=jax.ShapeDtypeStruct((_N, _C), jnp.float32),
    )(hd, p0, p1, Wsd, Wn)


def _head_body(hs_ref, m0_ref, m1_ref, n0_ref, n1_ref, Wss0_ref, Wnd0_ref,
               Wss1_ref, Wnd1_ref, Wh_ref, bh_ref, out_ref):
    f32 = jnp.float32
    m = m0_ref[...] + m1_ref[...]
    h2 = jnp.dot(hs_ref[...], Wss0_ref[...], preferred_element_type=f32)
    h2 = jnp.maximum(h2 + jnp.dot(m, Wnd0_ref[...],
                                  preferred_element_type=f32), 0.0)
    n = n0_ref[...] + n1_ref[...]
    h3 = jnp.dot(h2, Wss1_ref[...], preferred_element_type=f32)
    h3 = h3 + jnp.dot(n, Wnd1_ref[...], preferred_element_type=f32)
    out_ref[...] = jnp.dot(h3, Wh_ref[...],
                           preferred_element_type=f32) + bh_ref[...]


def _head(hs, msrc_flat, msrc2_flat, Wss0, Wnd0, Wss1, Wnd1, Wh, bh):
    top = pl.BlockSpec((_SEEDS, _C), lambda i: (0, 0))
    bot = pl.BlockSpec((_SEEDS, _C), lambda i: (1, 0))
    sq = pl.BlockSpec((_C, _C), lambda i: (0, 0))
    return pl.pallas_call(
        _head_body,
        grid=(1,),
        in_specs=[
            top, top, bot, top, bot, sq, sq, sq, sq,
            pl.BlockSpec((_C, _NUM_DST), lambda i: (0, 0)),
            pl.BlockSpec((1, _NUM_DST), lambda i: (0, 0)),
        ],
        out_specs=pl.BlockSpec((_SEEDS, _NUM_DST), lambda i: (0, 0)),
        out_shape=jax.ShapeDtypeStruct((_SEEDS, _NUM_DST), jnp.float32),
    )(hs, msrc_flat, msrc_flat, msrc2_flat, msrc2_flat,
      Wss0, Wnd0, Wss1, Wnd1, Wh, bh)


def kernel(x_src, x_dst, seed_time, node_time_src, W_enc_src, b_enc_src,
           W_enc_dst, b_enc_dst, W_time, b_time, W_self_src_0, W_neigh_ds_0,
           W_self_dst_0, W_neigh_sd_0, W_self_src_1, W_neigh_ds_1,
           W_self_dst_1, W_neigh_sd_1, W_head, b_head, edge_index_sd,
           edge_index_ds, batch_src):
    f32 = jnp.float32
    sd = edge_index_sd.astype(jnp.int32)
    ds = edge_index_ds.astype(jnp.int32)
    sd0, sd1 = sd[0], sd[1]
    ds0, ds1 = ds[0], ds[1]
    batch_col = batch_src.astype(jnp.int32).reshape(_N // _BR, _BR, 1)
    nt_col = node_time_src.reshape(_N // _BR, _BR, 1)
    seed_row = seed_time.reshape(1, _SEEDS)
    freqs = (2.0 ** jnp.arange(_NF, dtype=f32)).reshape(1, _NF)

    hs, hd = _encoder(x_src, x_dst, batch_col, nt_col, seed_row, freqs,
                      W_enc_src, b_enc_src.reshape(1, _C),
                      W_enc_dst, b_enc_dst.reshape(1, _C),
                      W_time, b_time.reshape(1, _C))

    mdst_flat, msrc_flat = _segsum0_kernel(hs, hd, sd0, sd1, ds0, ds1)
    p0 = mdst_flat[:_N]
    p1 = mdst_flat[_ACC_DST_ROWS:_ACC_DST_ROWS + _N]
    hd2 = _layer0_dst(hd, p0, p1, W_self_dst_0, W_neigh_sd_0)
    msrc2_flat = _segsum1_kernel(hd2, ds0, ds1)
    out = _head(hs, msrc_flat, msrc2_flat, W_self_src_0, W_neigh_ds_0,
                W_self_src_1, W_neigh_ds_1, W_head,
                b_head.reshape(1, _NUM_DST))
    return out


# speedup vs baseline: 8.0959x; 1.0961x over previous
import functools

import jax
import jax.numpy as jnp
from jax import lax
from jax.experimental import pallas as pl
from jax.experimental.pallas import tpu as pltpu
from jax.experimental.pallas import tpu_sc as plsc

_N = 10000
_E = 320000
_C = 128
_SEEDS = 512
_NF = 16
_NUM_DST = 10000

_NC = 2
_NS = 16
_NW = _NC * _NS

_K = 80
_TILE_EDGES = _E // _NW
_TILE_BATCHES = _TILE_EDGES // _K

_TRASH = _SEEDS
_ACC_SRC_ROWS = 640
_ACC_DST_ROWS = 10240

_BR = 1000

_mesh = functools.partial(
    plsc.VectorSubcoreMesh, core_axis_name="c", subcore_axis_name="s")


def _zero_rows(buf, nrows):
    zv = jnp.zeros((16,), jnp.float32)

    def body(r, carry):
        for k in range(_C // 16):
            buf[r, pl.ds(k * 16, 16)] = zv
        return carry

    lax.fori_loop(0, nrows, body, 0)


_CHUNK_E = 2000
_CHUNK_BATCHES = _CHUNK_E // _K
_COMP_WORDS = _CHUNK_E + _K


def _stage_batch(src_v, dst_ref, base):
    for k in range(_K // 16):
        dst_ref[pl.ds(k * 16, 16)] = src_v[pl.ds(base + k * 16, 16)]


def _db_pass(tbl_hbm, i0_hbm, i1_hbm, acc, idx0_v, idx1_v, ir0, iw0, ir1,
             iw1, rows0, rows1, sem0, sem1, wid, clamp):
    if clamp is not None:
        cl = jnp.full((16,), clamp, jnp.int32)
        lane = lax.iota(jnp.int32, 16)

    def stage(b, ir, iw):
        for k in range(_K // 16):
            sl = pl.ds(k * 16, 16)
            ir[sl] = idx0_v[pl.ds(b * _K + k * 16, 16)]
            seg = idx1_v[pl.ds(b * _K + k * 16, 16)]
            if clamp is None:
                iw[sl] = seg
            else:
                iw[sl] = jnp.where(seg < cl, seg, clamp + k * 16 + lane)

    def chunk_body(ch, carry):
        e0 = wid * _TILE_EDGES + ch * _CHUNK_E
        pltpu.sync_copy(i0_hbm.at[pl.ds(e0, _CHUNK_E)], idx0_v)
        pltpu.sync_copy(i1_hbm.at[pl.ds(e0, _CHUNK_E)], idx1_v)
        stage(0, ir0, iw0)
        cp = pltpu.async_copy(tbl_hbm.at[ir0], rows0, sem0)
        prev = (cp, iw0, rows0)
        for b in range(1, _CHUNK_BATCHES):
            if b % 2:
                ir, iw, rows, sem = ir1, iw1, rows1, sem1
            else:
                ir, iw, rows, sem = ir0, iw0, rows0, sem0
            stage(b, ir, iw)
            cp = pltpu.async_copy(tbl_hbm.at[ir], rows, sem)
            pcp, piw, prows = prev
            pcp.wait()
            pltpu.sync_copy(prows, acc.at[piw], add=True)
            prev = (cp, iw, rows)
        pcp, piw, prows = prev
        pcp.wait()
        pltpu.sync_copy(prows, acc.at[piw], add=True)
        return carry

    lax.fori_loop(0, _TILE_EDGES // _CHUNK_E, chunk_body, 0)


_SCANC = 400
_COMP_CAP = 480


def _filtered_pass(tbl_hbm, i0_hbm, i1_hbm, acc, idx0_v, idx1_v, ir0, iw0,
                   comp_s, comp_d, rows0, sem0, wid):
    lane = lax.iota(jnp.int32, 16)

    def build_flush(nb, cnt, final):
        def bbody(b, cr):
            for k in range(_K // 16):
                vs = jnp.zeros((16,), jnp.int32)
                vd = jnp.zeros((16,), jnp.int32)
                for t in range(16):
                    p = b * _K + k * 16 + t
                    s_sc = comp_s[p]
                    d_sc = comp_d[p]
                    if final:
                        ok = p < cnt
                        s_sc = jnp.where(ok, s_sc, 0)
                        d_sc = jnp.where(ok, d_sc,
                                         jnp.int32(_TRASH + k * 16 + t))
                    vs = jnp.where(lane == t, s_sc, vs)
                    vd = jnp.where(lane == t, d_sc, vd)
                ir0[pl.ds(k * 16, 16)] = vs
                iw0[pl.ds(k * 16, 16)] = vd
            pltpu.async_copy(tbl_hbm.at[ir0], rows0, sem0).wait()
            pltpu.sync_copy(rows0, acc.at[iw0], add=True)
            return cr

        lax.fori_loop(0, nb, bbody, 0)

    def chunk_body(ch, cnt):
        e0 = wid * _TILE_EDGES + ch * _CHUNK_E
        pltpu.sync_copy(i0_hbm.at[pl.ds(e0, _CHUNK_E)], idx0_v)
        pltpu.sync_copy(i1_hbm.at[pl.ds(e0, _CHUNK_E)], idx1_v)

        def sub(j, cnt2):
            def s16(i, cnt3):
                off = j * _SCANC + i * 16
                dv = idx1_v[pl.ds(off, 16)]
                sv = idx0_v[pl.ds(off, 16)]
                for t in range(16):
                    d = dv[t]
                    s0 = sv[t]
                    comp_s[cnt3] = s0
                    comp_d[cnt3] = d
                    cnt3 = jnp.where(d < _SEEDS, cnt3 + 1, cnt3)
                return cnt3

            cnt2 = lax.fori_loop(0, _SCANC // 16, s16, cnt2)
            nb = cnt2 // _K
            build_flush(nb, cnt2, False)

            def mv(t, cr):
                comp_s[t] = comp_s[nb * _K + t]
                comp_d[t] = comp_d[nb * _K + t]
                return cr

            lax.fori_loop(0, _K, mv, 0)
            return cnt2 - nb * _K

        return lax.fori_loop(0, _CHUNK_E // _SCANC, sub, cnt)

    cnt = lax.fori_loop(0, _TILE_EDGES // _CHUNK_E, chunk_body, jnp.int32(0))
    build_flush((cnt + _K - 1) // _K, cnt, True)


@functools.partial(
    pl.kernel,
    mesh=_mesh(),
    out_type=[
        jax.ShapeDtypeStruct((2 * _ACC_DST_ROWS, _C), jnp.float32),
        jax.ShapeDtypeStruct((2 * _SEEDS, _C), jnp.float32),
    ],
    scratch_types=[
        pltpu.VMEM((_CHUNK_E,), jnp.int32),
        pltpu.VMEM((_CHUNK_E,), jnp.int32),
        pltpu.VMEM((_K,), jnp.int32),
        pltpu.VMEM((_K,), jnp.int32),
        pltpu.VMEM((_K,), jnp.int32),
        pltpu.VMEM((_K,), jnp.int32),
        pltpu.VMEM((_K, _C), jnp.float32),
        pltpu.VMEM((_K, _C), jnp.float32),
        pltpu.VMEM((32, _C), jnp.float32),
        pltpu.SMEM((_COMP_CAP,), jnp.int32),
        pltpu.SMEM((_COMP_CAP,), jnp.int32),
        pltpu.VMEM_SHARED((_ACC_DST_ROWS, _C), jnp.float32),
        pltpu.VMEM_SHARED((_ACC_SRC_ROWS, _C), jnp.float32),
        pltpu.SemaphoreType.DMA,
        pltpu.SemaphoreType.DMA,
    ],
)
def _segsum0_kernel(hsrc, hdst, sd0, sd1, ds0, ds1, mdst_out, msrc_out,
                    idx0_v, idx1_v, ir0, iw0, ir1, iw1,
                    rows0, rows1, zbuf, comp_s, comp_d, acc_dst, acc_src,
                    sem0, sem1):
    c = lax.axis_index("c")
    s = lax.axis_index("s")
    wid = s * _NC + c
    _zero_rows(zbuf, 32)
    for i in range(20):
        pltpu.sync_copy(zbuf, acc_dst.at[pl.ds(s * 640 + i * 32, 32)])
    pltpu.sync_copy(zbuf, acc_src.at[pl.ds(s * 40, 32)])
    pltpu.sync_copy(zbuf.at[pl.ds(0, 8)], acc_src.at[pl.ds(s * 40 + 32, 8)])
    plsc.subcore_barrier()
    _db_pass(hsrc, sd0, sd1, acc_dst, idx0_v, idx1_v, ir0, iw0, ir1, iw1,
             rows0, rows1, sem0, sem1, wid, None)
    _filtered_pass(hdst, ds0, ds1, acc_src, idx0_v, idx1_v, ir0, iw0,
                   comp_s, comp_d, rows0, sem0, wid)
    plsc.subcore_barrier()
    for i in range(20):
        r0 = s * 640 + i * 32
        pltpu.sync_copy(acc_dst.at[pl.ds(r0, 32)], zbuf)
        pltpu.sync_copy(zbuf, mdst_out.at[pl.ds(c * _ACC_DST_ROWS + r0, 32)])
    pltpu.sync_copy(acc_src.at[pl.ds(s * 32, 32)], zbuf)
    pltpu.sync_copy(zbuf, msrc_out.at[pl.ds(c * _SEEDS + s * 32, 32)])


@functools.partial(
    pl.kernel,
    mesh=_mesh(),
    out_type=jax.ShapeDtypeStruct((2 * _SEEDS, _C), jnp.float32),
    scratch_types=[
        pltpu.VMEM((_CHUNK_E,), jnp.int32),
        pltpu.VMEM((_CHUNK_E,), jnp.int32),
        pltpu.VMEM((_K,), jnp.int32),
        pltpu.VMEM((_K,), jnp.int32),
        pltpu.VMEM((_K, _C), jnp.float32),
        pltpu.VMEM((40, _C), jnp.float32),
        pltpu.SMEM((_COMP_CAP,), jnp.int32),
        pltpu.SMEM((_COMP_CAP,), jnp.int32),
        pltpu.VMEM_SHARED((_ACC_SRC_ROWS, _C), jnp.float32),
        pltpu.SemaphoreType.DMA,
    ],
)
def _segsum1_kernel(hdst2, ds0, ds1, msrc2_out, idx0_v, idx1_v, ir0, iw0,
                    rows0, zbuf, comp_s, comp_d, acc_src, sem0):
    c = lax.axis_index("c")
    s = lax.axis_index("s")
    wid = s * _NC + c
    _zero_rows(zbuf, 40)
    pltpu.sync_copy(zbuf, acc_src.at[pl.ds(s * 40, 40)])
    plsc.subcore_barrier()
    _filtered_pass(hdst2, ds0, ds1, acc_src, idx0_v, idx1_v, ir0, iw0,
                   comp_s, comp_d, rows0, sem0, wid)
    plsc.subcore_barrier()
    pltpu.sync_copy(acc_src.at[pl.ds(s * 32, 32)], zbuf.at[pl.ds(0, 32)])
    pltpu.sync_copy(zbuf.at[pl.ds(0, 32)],
                    msrc2_out.at[pl.ds(c * _SEEDS + s * 32, 32)])


def _enc_body(xs_ref, xd_ref, batch_ref, nt_ref, st_ref, fr_ref, Wes_ref,
              bes_ref, Wed_ref, bed_ref, Wt_ref, bt_ref, hs_ref, hd_ref):
    f32 = jnp.float32
    hs = jnp.dot(xs_ref[...], Wes_ref[...], preferred_element_type=f32)
    hs = hs + bes_ref[...]
    batch = batch_ref[...][0]
    oh = batch == lax.broadcasted_iota(jnp.int32, (_BR, _SEEDS), 1)
    seedg = jnp.sum(jnp.where(oh, st_ref[...], 0.0), axis=1, keepdims=True)
    rel = seedg - nt_ref[...][0]
    ang = rel * fr_ref[...]
    pe = jnp.concatenate([jnp.sin(ang), jnp.cos(ang)], axis=1)
    hs = hs + jnp.dot(pe, Wt_ref[...], preferred_element_type=f32)
    hs_ref[...] = hs + bt_ref[...]
    hd = jnp.dot(xd_ref[...], Wed_ref[...], preferred_element_type=f32)
    hd_ref[...] = hd + bed_ref[...]


def _encoder(x_src, x_dst, batch_col, nt_col, seed_row, freqs,
             Wes, bes, Wed, bed, Wt, bt):
    nb = _N // _BR
    row = pl.BlockSpec((_BR, _C), lambda i: (i, 0))
    col = pl.BlockSpec((1, _BR, 1), lambda i: (i, 0, 0))
    return pl.pallas_call(
        _enc_body,
        grid=(nb,),
        in_specs=[
            row,
            row,
            col,
            col,
            pl.BlockSpec((1, _SEEDS), lambda i: (0, 0)),
            pl.BlockSpec((1, _NF), lambda i: (0, 0)),
            pl.BlockSpec((_C, _C), lambda i: (0, 0)),
            pl.BlockSpec((1, _C), lambda i: (0, 0)),
            pl.BlockSpec((_C, _C), lambda i: (0, 0)),
            pl.BlockSpec((1, _C), lambda i: (0, 0)),
            pl.BlockSpec((2 * _NF, _C), lambda i: (0, 0)),
            pl.BlockSpec((1, _C), lambda i: (0, 0)),
        ],
        out_specs=[row, row],
        out_shape=[
            jax.ShapeDtypeStruct((_N, _C), jnp.float32),
            jax.ShapeDtypeStruct((_N, _C), jnp.float32),
        ],
    )(x_src, x_dst, batch_col, nt_col, seed_row, freqs,
      Wes, bes, Wed, bed, Wt, bt)


def _l0_body(hd_ref, m0_ref, m1_ref, Wsd_ref, Wn_ref, out_ref):
    f32 = jnp.float32
    m = m0_ref[...] + m1_ref[...]
    x = jnp.dot(hd_ref[...], Wsd_ref[...], preferred_element_type=f32)
    x = x + jnp.dot(m, Wn_ref[...], preferred_element_type=f32)
    out_ref[...] = jnp.maximum(x, 0.0)


def _layer0_dst(hd, p0, p1, Wsd, Wn):
    nb = _N // _BR
    row = pl.BlockSpec((_BR, _C), lambda i: (i, 0))
    sq = pl.BlockSpec((_C, _C), lambda i: (0, 0))
    return pl.pallas_call(
        _l0_body,
        grid=(nb,),
        in_specs=[row, row, row, sq, sq],
        out_specs=row,
        out_shape=jax.ShapeDtypeStruct((_N, _C), jnp.float32),
    )(hd, p0, p1, Wsd, Wn)


def _head_body(hs_ref, m0_ref, m1_ref, n0_ref, n1_ref, Wss0_ref, Wnd0_ref,
               Wss1_ref, Wnd1_ref, Wh_ref, bh_ref, out_ref):
    f32 = jnp.float32
    m = m0_ref[...] + m1_ref[...]
    h2 = jnp.dot(hs_ref[...], Wss0_ref[...], preferred_element_type=f32)
    h2 = jnp.maximum(h2 + jnp.dot(m, Wnd0_ref[...],
                                  preferred_element_type=f32), 0.0)
    n = n0_ref[...] + n1_ref[...]
    h3 = jnp.dot(h2, Wss1_ref[...], preferred_element_type=f32)
    h3 = h3 + jnp.dot(n, Wnd1_ref[...], preferred_element_type=f32)
    out_ref[...] = jnp.dot(h3, Wh_ref[...],
                           preferred_element_type=f32) + bh_ref[...]


def _head(hs, msrc_flat, msrc2_flat, Wss0, Wnd0, Wss1, Wnd1, Wh, bh):
    top = pl.BlockSpec((_SEEDS, _C), lambda i: (0, 0))
    bot = pl.BlockSpec((_SEEDS, _C), lambda i: (1, 0))
    sq = pl.BlockSpec((_C, _C), lambda i: (0, 0))
    return pl.pallas_call(
        _head_body,
        grid=(1,),
        in_specs=[
            top, top, bot, top, bot, sq, sq, sq, sq,
            pl.BlockSpec((_C, _NUM_DST), lambda i: (0, 0)),
            pl.BlockSpec((1, _NUM_DST), lambda i: (0, 0)),
        ],
        out_specs=pl.BlockSpec((_SEEDS, _NUM_DST), lambda i: (0, 0)),
        out_shape=jax.ShapeDtypeStruct((_SEEDS, _NUM_DST), jnp.float32),
    )(hs, msrc_flat, msrc_flat, msrc2_flat, msrc2_flat,
      Wss0, Wnd0, Wss1, Wnd1, Wh, bh)


def kernel(x_src, x_dst, seed_time, node_time_src, W_enc_src, b_enc_src,
           W_enc_dst, b_enc_dst, W_time, b_time, W_self_src_0, W_neigh_ds_0,
           W_self_dst_0, W_neigh_sd_0, W_self_src_1, W_neigh_ds_1,
           W_self_dst_1, W_neigh_sd_1, W_head, b_head, edge_index_sd,
           edge_index_ds, batch_src):
    f32 = jnp.float32
    sd = edge_index_sd.astype(jnp.int32)
    ds = edge_index_ds.astype(jnp.int32)
    sd0, sd1 = sd[0], sd[1]
    ds0, ds1 = ds[0], ds[1]
    batch_col = batch_src.astype(jnp.int32).reshape(_N // _BR, _BR, 1)
    nt_col = node_time_src.reshape(_N // _BR, _BR, 1)
    seed_row = seed_time.reshape(1, _SEEDS)
    freqs = (2.0 ** jnp.arange(_NF, dtype=f32)).reshape(1, _NF)

    hs, hd = _encoder(x_src, x_dst, batch_col, nt_col, seed_row, freqs,
                      W_enc_src, b_enc_src.reshape(1, _C),
                      W_enc_dst, b_enc_dst.reshape(1, _C),
                      W_time, b_time.reshape(1, _C))

    mdst_flat, msrc_flat = _segsum0_kernel(hs, hd, sd0, sd1, ds0, ds1)
    p0 = mdst_flat[:_N]
    p1 = mdst_flat[_ACC_DST_ROWS:_ACC_DST_ROWS + _N]
    hd2 = _layer0_dst(hd, p0, p1, W_self_dst_0, W_neigh_sd_0)
    msrc2_flat = _segsum1_kernel(hd2, ds0, ds1)
    out = _head(hs, msrc_flat, msrc2_flat, W_self_src_0, W_neigh_ds_0,
                W_self_src_1, W_neigh_ds_1, W_head,
                b_head.reshape(1, _NUM_DST))
    return out


# speedup vs baseline: 8.5681x; 1.0583x over previous
import functools

import jax
import jax.numpy as jnp
from jax import lax
from jax.experimental import pallas as pl
from jax.experimental.pallas import tpu as pltpu
from jax.experimental.pallas import tpu_sc as plsc

_N = 10000
_E = 320000
_C = 128
_SEEDS = 512
_NF = 16
_NUM_DST = 10000

_NC = 2
_NS = 16
_NW = _NC * _NS

_K = 80
_TILE_EDGES = _E // _NW
_TILE_BATCHES = _TILE_EDGES // _K

_TRASH = _SEEDS
_ACC_SRC_ROWS = 640
_ACC_DST_ROWS = 10240

_BR = 1000

_mesh = functools.partial(
    plsc.VectorSubcoreMesh, core_axis_name="c", subcore_axis_name="s")


def _zero_rows(buf, nrows):
    zv = jnp.zeros((16,), jnp.float32)

    def body(r, carry):
        for k in range(_C // 16):
            buf[r, pl.ds(k * 16, 16)] = zv
        return carry

    lax.fori_loop(0, nrows, body, 0)


_CHUNK_E = 2000
_CHUNK_BATCHES = _CHUNK_E // _K
_COMP_WORDS = _CHUNK_E + _K


def _stage_batch(src_v, dst_ref, base):
    for k in range(_K // 16):
        dst_ref[pl.ds(k * 16, 16)] = src_v[pl.ds(base + k * 16, 16)]


def _db_pass(tbl_hbm, i0_hbm, i1_hbm, acc, idx0_v, idx1_v, ir0, iw0, ir1,
             iw1, rows0, rows1, sem0, sem1, wid, clamp):
    if clamp is not None:
        cl = jnp.full((16,), clamp, jnp.int32)
        lane = lax.iota(jnp.int32, 16)

    def stage(b, ir, iw):
        for k in range(_K // 16):
            sl = pl.ds(k * 16, 16)
            ir[sl] = idx0_v[pl.ds(b * _K + k * 16, 16)]
            seg = idx1_v[pl.ds(b * _K + k * 16, 16)]
            if clamp is None:
                iw[sl] = seg
            else:
                iw[sl] = jnp.where(seg < cl, seg, clamp + k * 16 + lane)

    def chunk_body(ch, carry):
        e0 = wid * _TILE_EDGES + ch * _CHUNK_E
        pltpu.sync_copy(i0_hbm.at[pl.ds(e0, _CHUNK_E)], idx0_v)
        pltpu.sync_copy(i1_hbm.at[pl.ds(e0, _CHUNK_E)], idx1_v)
        stage(0, ir0, iw0)
        cp = pltpu.async_copy(tbl_hbm.at[ir0], rows0, sem0)
        prev = (cp, iw0, rows0)
        for b in range(1, _CHUNK_BATCHES):
            if b % 2:
                ir, iw, rows, sem = ir1, iw1, rows1, sem1
            else:
                ir, iw, rows, sem = ir0, iw0, rows0, sem0
            stage(b, ir, iw)
            cp = pltpu.async_copy(tbl_hbm.at[ir], rows, sem)
            pcp, piw, prows = prev
            pcp.wait()
            pltpu.sync_copy(prows, acc.at[piw], add=True)
            prev = (cp, iw, rows)
        pcp, piw, prows = prev
        pcp.wait()
        pltpu.sync_copy(prows, acc.at[piw], add=True)
        return carry

    lax.fori_loop(0, _TILE_EDGES // _CHUNK_E, chunk_body, 0)


_SCANC = 400
_COMP_CAP = 480


def _filtered_pass(tbl_hbm, i0_hbm, i1_hbm, acc, idx0_v, idx1_v, ir0, iw0,
                   comp_p, rows0, sem0, wid):
    lane = lax.iota(jnp.int32, 16)
    cl = jnp.full((16,), _SEEDS, jnp.int32)

    def build_flush(nb, cnt, final):
        def bbody(b, cr):
            for k in range(_K // 16):
                vp = jnp.zeros((16,), jnp.int32)
                for t in range(16):
                    p = b * _K + k * 16 + t
                    pk = comp_p[p]
                    if final:
                        pk = jnp.where(p < cnt, pk, -1)
                    vp = jnp.where(lane == t, pk, vp)
                ok = vp >= 0
                ir0[pl.ds(k * 16, 16)] = jnp.where(
                    ok, lax.shift_right_logical(vp, 9), 0)
                iw0[pl.ds(k * 16, 16)] = jnp.where(
                    ok, vp & 511, _TRASH + k * 16 + lane)
            pltpu.async_copy(tbl_hbm.at[ir0], rows0, sem0).wait()
            pltpu.sync_copy(rows0, acc.at[iw0], add=True)
            return cr

        lax.fori_loop(0, nb, bbody, 0)

    def chunk_body(ch, cnt):
        e0 = wid * _TILE_EDGES + ch * _CHUNK_E
        pltpu.sync_copy(i0_hbm.at[pl.ds(e0, _CHUNK_E)], idx0_v)
        pltpu.sync_copy(i1_hbm.at[pl.ds(e0, _CHUNK_E)], idx1_v)

        def sub(j, cnt2):
            def s16(i, cnt3):
                off = j * _SCANC + i * 16
                dv = idx1_v[pl.ds(off, 16)]
                sv = idx0_v[pl.ds(off, 16)]
                pkv = jnp.where(dv < cl,
                                lax.shift_left(sv, 9) + dv, -1)
                for t in range(16):
                    pk = pkv[t]
                    comp_p[cnt3] = pk
                    cnt3 = jnp.where(pk >= 0, cnt3 + 1, cnt3)
                return cnt3

            cnt2 = lax.fori_loop(0, _SCANC // 16, s16, cnt2)
            nb = cnt2 // _K
            build_flush(nb, cnt2, False)

            def mv(t, cr):
                comp_p[t] = comp_p[nb * _K + t]
                return cr

            lax.fori_loop(0, _K, mv, 0)
            return cnt2 - nb * _K

        return lax.fori_loop(0, _CHUNK_E // _SCANC, sub, cnt)

    cnt = lax.fori_loop(0, _TILE_EDGES // _CHUNK_E, chunk_body, jnp.int32(0))
    build_flush((cnt + _K - 1) // _K, cnt, True)


@functools.partial(
    pl.kernel,
    mesh=_mesh(),
    out_type=[
        jax.ShapeDtypeStruct((2 * _ACC_DST_ROWS, _C), jnp.float32),
        jax.ShapeDtypeStruct((2 * _SEEDS, _C), jnp.float32),
    ],
    scratch_types=[
        pltpu.VMEM((_CHUNK_E,), jnp.int32),
        pltpu.VMEM((_CHUNK_E,), jnp.int32),
        pltpu.VMEM((_K,), jnp.int32),
        pltpu.VMEM((_K,), jnp.int32),
        pltpu.VMEM((_K,), jnp.int32),
        pltpu.VMEM((_K,), jnp.int32),
        pltpu.VMEM((_K, _C), jnp.float32),
        pltpu.VMEM((_K, _C), jnp.float32),
        pltpu.VMEM((32, _C), jnp.float32),
        pltpu.SMEM((_COMP_CAP,), jnp.int32),
        pltpu.VMEM_SHARED((_ACC_DST_ROWS, _C), jnp.float32),
        pltpu.VMEM_SHARED((_ACC_SRC_ROWS, _C), jnp.float32),
        pltpu.SemaphoreType.DMA,
        pltpu.SemaphoreType.DMA,
    ],
)
def _segsum0_kernel(hsrc, hdst, sd0, sd1, ds0, ds1, mdst_out, msrc_out,
                    idx0_v, idx1_v, ir0, iw0, ir1, iw1,
                    rows0, rows1, zbuf, comp_p, acc_dst, acc_src,
                    sem0, sem1):
    c = lax.axis_index("c")
    s = lax.axis_index("s")
    wid = s * _NC + c
    _zero_rows(zbuf, 32)
    for i in range(20):
        pltpu.sync_copy(zbuf, acc_dst.at[pl.ds(s * 640 + i * 32, 32)])
    pltpu.sync_copy(zbuf, acc_src.at[pl.ds(s * 40, 32)])
    pltpu.sync_copy(zbuf.at[pl.ds(0, 8)], acc_src.at[pl.ds(s * 40 + 32, 8)])
    plsc.subcore_barrier()
    _db_pass(hsrc, sd0, sd1, acc_dst, idx0_v, idx1_v, ir0, iw0, ir1, iw1,
             rows0, rows1, sem0, sem1, wid, None)
    _filtered_pass(hdst, ds0, ds1, acc_src, idx0_v, idx1_v, ir0, iw0,
                   comp_p, rows0, sem0, wid)
    plsc.subcore_barrier()
    for i in range(20):
        r0 = s * 640 + i * 32
        pltpu.sync_copy(acc_dst.at[pl.ds(r0, 32)], zbuf)
        pltpu.sync_copy(zbuf, mdst_out.at[pl.ds(c * _ACC_DST_ROWS + r0, 32)])
    pltpu.sync_copy(acc_src.at[pl.ds(s * 32, 32)], zbuf)
    pltpu.sync_copy(zbuf, msrc_out.at[pl.ds(c * _SEEDS + s * 32, 32)])


@functools.partial(
    pl.kernel,
    mesh=_mesh(),
    out_type=jax.ShapeDtypeStruct((2 * _SEEDS, _C), jnp.float32),
    scratch_types=[
        pltpu.VMEM((_CHUNK_E,), jnp.int32),
        pltpu.VMEM((_CHUNK_E,), jnp.int32),
        pltpu.VMEM((_K,), jnp.int32),
        pltpu.VMEM((_K,), jnp.int32),
        pltpu.VMEM((_K, _C), jnp.float32),
        pltpu.VMEM((40, _C), jnp.float32),
        pltpu.SMEM((_COMP_CAP,), jnp.int32),
        pltpu.VMEM_SHARED((_ACC_SRC_ROWS, _C), jnp.float32),
        pltpu.SemaphoreType.DMA,
    ],
)
def _segsum1_kernel(hdst2, ds0, ds1, msrc2_out, idx0_v, idx1_v, ir0, iw0,
                    rows0, zbuf, comp_p, acc_src, sem0):
    c = lax.axis_index("c")
    s = lax.axis_index("s")
    wid = s * _NC + c
    _zero_rows(zbuf, 40)
    pltpu.sync_copy(zbuf, acc_src.at[pl.ds(s * 40, 40)])
    plsc.subcore_barrier()
    _filtered_pass(hdst2, ds0, ds1, acc_src, idx0_v, idx1_v, ir0, iw0,
                   comp_p, rows0, sem0, wid)
    plsc.subcore_barrier()
    pltpu.sync_copy(acc_src.at[pl.ds(s * 32, 32)], zbuf.at[pl.ds(0, 32)])
    pltpu.sync_copy(zbuf.at[pl.ds(0, 32)],
                    msrc2_out.at[pl.ds(c * _SEEDS + s * 32, 32)])


def _enc_body(xs_ref, xd_ref, batch_ref, nt_ref, st_ref, fr_ref, Wes_ref,
              bes_ref, Wed_ref, bed_ref, Wt_ref, bt_ref, hs_ref, hd_ref):
    f32 = jnp.float32
    hs = jnp.dot(xs_ref[...], Wes_ref[...], preferred_element_type=f32)
    hs = hs + bes_ref[...]
    batch = batch_ref[...][0]
    oh = batch == lax.broadcasted_iota(jnp.int32, (_BR, _SEEDS), 1)
    seedg = jnp.sum(jnp.where(oh, st_ref[...], 0.0), axis=1, keepdims=True)
    rel = seedg - nt_ref[...][0]
    ang = rel * fr_ref[...]
    pe = jnp.concatenate([jnp.sin(ang), jnp.cos(ang)], axis=1)
    hs = hs + jnp.dot(pe, Wt_ref[...], preferred_element_type=f32)
    hs_ref[...] = hs + bt_ref[...]
    hd = jnp.dot(xd_ref[...], Wed_ref[...], preferred_element_type=f32)
    hd_ref[...] = hd + bed_ref[...]


def _encoder(x_src, x_dst, batch_col, nt_col, seed_row, freqs,
             Wes, bes, Wed, bed, Wt, bt):
    nb = _N // _BR
    row = pl.BlockSpec((_BR, _C), lambda i: (i, 0))
    col = pl.BlockSpec((1, _BR, 1), lambda i: (i, 0, 0))
    return pl.pallas_call(
        _enc_body,
        grid=(nb,),
        in_specs=[
            row,
            row,
            col,
            col,
            pl.BlockSpec((1, _SEEDS), lambda i: (0, 0)),
            pl.BlockSpec((1, _NF), lambda i: (0, 0)),
            pl.BlockSpec((_C, _C), lambda i: (0, 0)),
            pl.BlockSpec((1, _C), lambda i: (0, 0)),
            pl.BlockSpec((_C, _C), lambda i: (0, 0)),
            pl.BlockSpec((1, _C), lambda i: (0, 0)),
            pl.BlockSpec((2 * _NF, _C), lambda i: (0, 0)),
            pl.BlockSpec((1, _C), lambda i: (0, 0)),
        ],
        out_specs=[row, row],
        out_shape=[
            jax.ShapeDtypeStruct((_N, _C), jnp.float32),
            jax.ShapeDtypeStruct((_N, _C), jnp.float32),
        ],
    )(x_src, x_dst, batch_col, nt_col, seed_row, freqs,
      Wes, bes, Wed, bed, Wt, bt)


def _l0_body(hd_ref, m0_ref, m1_ref, Wsd_ref, Wn_ref, out_ref):
    f32 = jnp.float32
    m = m0_ref[...] + m1_ref[...]
    x = jnp.dot(hd_ref[...], Wsd_ref[...], preferred_element_type=f32)
    x = x + jnp.dot(m, Wn_ref[...], preferred_element_type=f32)
    out_ref[...] = jnp.maximum(x, 0.0)


def _layer0_dst(hd, p0, p1, Wsd, Wn):
    nb = _N // _BR
    row = pl.BlockSpec((_BR, _C), lambda i: (i, 0))
    sq = pl.BlockSpec((_C, _C), lambda i: (0, 0))
    return pl.pallas_call(
        _l0_body,
        grid=(nb,),
        in_specs=[row, row, row, sq, sq],
        out_specs=row,
        out_shape=jax.ShapeDtypeStruct((_N, _C), jnp.float32),
    )(hd, p0, p1, Wsd, Wn)


def _head_body(hs_ref, m0_ref, m1_ref, n0_ref, n1_ref, Wss0_ref, Wnd0_ref,
               Wss1_ref, Wnd1_ref, Wh_ref, bh_ref, out_ref):
    f32 = jnp.float32
    m = m0_ref[...] + m1_ref[...]
    h2 = jnp.dot(hs_ref[...], Wss0_ref[...], preferred_element_type=f32)
    h2 = jnp.maximum(h2 + jnp.dot(m, Wnd0_ref[...],
                                  preferred_element_type=f32), 0.0)
    n = n0_ref[...] + n1_ref[...]
    h3 = jnp.dot(h2, Wss1_ref[...], preferred_element_type=f32)
    h3 = h3 + jnp.dot(n, Wnd1_ref[...], preferred_element_type=f32)
    out_ref[...] = jnp.dot(h3, Wh_ref[...],
                           preferred_element_type=f32) + bh_ref[...]


def _head(hs, msrc_flat, msrc2_flat, Wss0, Wnd0, Wss1, Wnd1, Wh, bh):
    top = pl.BlockSpec((_SEEDS, _C), lambda i: (0, 0))
    bot = pl.BlockSpec((_SEEDS, _C), lambda i: (1, 0))
    sq = pl.BlockSpec((_C, _C), lambda i: (0, 0))
    return pl.pallas_call(
        _head_body,
        grid=(1,),
        in_specs=[
            top, top, bot, top, bot, sq, sq, sq, sq,
            pl.BlockSpec((_C, _NUM_DST), lambda i: (0, 0)),
            pl.BlockSpec((1, _NUM_DST), lambda i: (0, 0)),
        ],
        out_specs=pl.BlockSpec((_SEEDS, _NUM_DST), lambda i: (0, 0)),
        out_shape=jax.ShapeDtypeStruct((_SEEDS, _NUM_DST), jnp.float32),
    )(hs, msrc_flat, msrc_flat, msrc2_flat, msrc2_flat,
      Wss0, Wnd0, Wss1, Wnd1, Wh, bh)


def kernel(x_src, x_dst, seed_time, node_time_src, W_enc_src, b_enc_src,
           W_enc_dst, b_enc_dst, W_time, b_time, W_self_src_0, W_neigh_ds_0,
           W_self_dst_0, W_neigh_sd_0, W_self_src_1, W_neigh_ds_1,
           W_self_dst_1, W_neigh_sd_1, W_head, b_head, edge_index_sd,
           edge_index_ds, batch_src):
    f32 = jnp.float32
    sd = edge_index_sd.astype(jnp.int32)
    ds = edge_index_ds.astype(jnp.int32)
    sd0, sd1 = sd[0], sd[1]
    ds0, ds1 = ds[0], ds[1]
    batch_col = batch_src.astype(jnp.int32).reshape(_N // _BR, _BR, 1)
    nt_col = node_time_src.reshape(_N // _BR, _BR, 1)
    seed_row = seed_time.reshape(1, _SEEDS)
    freqs = (2.0 ** jnp.arange(_NF, dtype=f32)).reshape(1, _NF)

    hs, hd = _encoder(x_src, x_dst, batch_col, nt_col, seed_row, freqs,
                      W_enc_src, b_enc_src.reshape(1, _C),
                      W_enc_dst, b_enc_dst.reshape(1, _C),
                      W_time, b_time.reshape(1, _C))

    mdst_flat, msrc_flat = _segsum0_kernel(hs, hd, sd0, sd1, ds0, ds1)
    p0 = mdst_flat[:_N]
    p1 = mdst_flat[_ACC_DST_ROWS:_ACC_DST_ROWS + _N]
    hd2 = _layer0_dst(hd, p0, p1, W_self_dst_0, W_neigh_sd_0)
    msrc2_flat = _segsum1_kernel(hd2, ds0, ds1)
    out = _head(hs, msrc_flat, msrc2_flat, W_self_src_0, W_neigh_ds_0,
                W_self_src_1, W_neigh_ds_1, W_head,
                b_head.reshape(1, _NUM_DST))
    return out


# speedup vs baseline: 8.6063x; 1.0045x over previous
import functools

import jax
import jax.numpy as jnp
from jax import lax
from jax.experimental import pallas as pl
from jax.experimental.pallas import tpu as pltpu
from jax.experimental.pallas import tpu_sc as plsc

_N = 10000
_E = 320000
_C = 128
_SEEDS = 512
_NF = 16
_NUM_DST = 10000

_NC = 2
_NS = 16
_NW = _NC * _NS

_K = 80
_TILE_EDGES = _E // _NW
_TILE_BATCHES = _TILE_EDGES // _K

_TRASH = _SEEDS
_ACC_SRC_ROWS = 640
_ACC_DST_ROWS = 10240

_BR = 1000

_mesh = functools.partial(
    plsc.VectorSubcoreMesh, core_axis_name="c", subcore_axis_name="s")


def _zero_rows(buf, nrows):
    zv = jnp.zeros((16,), jnp.float32)

    def body(r, carry):
        for k in range(_C // 16):
            buf[r, pl.ds(k * 16, 16)] = zv
        return carry

    lax.fori_loop(0, nrows, body, 0)


_CHUNK_E = 2000
_CHUNK_BATCHES = _CHUNK_E // _K
_COMP_WORDS = _CHUNK_E + _K


def _stage_batch(src_v, dst_ref, base):
    for k in range(_K // 16):
        dst_ref[pl.ds(k * 16, 16)] = src_v[pl.ds(base + k * 16, 16)]


def _db_pass(tbl_hbm, i0_hbm, i1_hbm, acc, idx0_v, idx1_v, ir0, iw0, ir1,
             iw1, rows0, rows1, sem0, sem1, wid, clamp):
    if clamp is not None:
        cl = jnp.full((16,), clamp, jnp.int32)
        lane = lax.iota(jnp.int32, 16)

    def stage(b, ir, iw):
        for k in range(_K // 16):
            sl = pl.ds(k * 16, 16)
            ir[sl] = idx0_v[pl.ds(b * _K + k * 16, 16)]
            seg = idx1_v[pl.ds(b * _K + k * 16, 16)]
            if clamp is None:
                iw[sl] = seg
            else:
                iw[sl] = jnp.where(seg < cl, seg, clamp + k * 16 + lane)

    def chunk_body(ch, carry):
        e0 = wid * _TILE_EDGES + ch * _CHUNK_E
        pltpu.sync_copy(i0_hbm.at[pl.ds(e0, _CHUNK_E)], idx0_v)
        pltpu.sync_copy(i1_hbm.at[pl.ds(e0, _CHUNK_E)], idx1_v)
        stage(0, ir0, iw0)
        cp = pltpu.async_copy(tbl_hbm.at[ir0], rows0, sem0)
        prev = (cp, iw0, rows0)
        for b in range(1, _CHUNK_BATCHES):
            if b % 2:
                ir, iw, rows, sem = ir1, iw1, rows1, sem1
            else:
                ir, iw, rows, sem = ir0, iw0, rows0, sem0
            stage(b, ir, iw)
            cp = pltpu.async_copy(tbl_hbm.at[ir], rows, sem)
            pcp, piw, prows = prev
            pcp.wait()
            pltpu.sync_copy(prows, acc.at[piw], add=True)
            prev = (cp, iw, rows)
        pcp, piw, prows = prev
        pcp.wait()
        pltpu.sync_copy(prows, acc.at[piw], add=True)
        return carry

    lax.fori_loop(0, _TILE_EDGES // _CHUNK_E, chunk_body, 0)


_SCANC = 400
_COMP_CAP = 480


def _filtered_pass(tbl_hbm, i0_hbm, i1_hbm, acc, idx0_v, idx1_v, ir0, iw0,
                   comp_p, rows0, sem0, wid):
    lane = lax.iota(jnp.int32, 16)
    cl = jnp.full((16,), _SEEDS, jnp.int32)

    def build_flush(nb, cnt, final):
        def bbody(b, cr):
            for k in range(_K // 16):
                vp = jnp.zeros((16,), jnp.int32)
                for t in range(16):
                    p = b * _K + k * 16 + t
                    pk = comp_p[p]
                    if final:
                        pk = jnp.where(p < cnt, pk, -1)
                    vp = jnp.where(lane == t, pk, vp)
                ok = vp >= 0
                ir0[pl.ds(k * 16, 16)] = jnp.where(
                    ok, lax.shift_right_logical(vp, 9), 0)
                iw0[pl.ds(k * 16, 16)] = jnp.where(
                    ok, vp & 511, _TRASH + k * 16 + lane)
            pltpu.async_copy(tbl_hbm.at[ir0], rows0, sem0).wait()
            pltpu.sync_copy(rows0, acc.at[iw0], add=True)
            return cr

        lax.fori_loop(0, nb, bbody, 0)

    def chunk_body(ch, cnt):
        e0 = wid * _TILE_EDGES + ch * _CHUNK_E
        pltpu.sync_copy(i0_hbm.at[pl.ds(e0, _CHUNK_E)], idx0_v)
        pltpu.sync_copy(i1_hbm.at[pl.ds(e0, _CHUNK_E)], idx1_v)

        def sub(j, cnt2):
            def s16(i, cnt3):
                off = j * _SCANC + i * 16
                dv = idx1_v[pl.ds(off, 16)]
                sv = idx0_v[pl.ds(off, 16)]
                pkv = jnp.where(dv < cl,
                                lax.shift_left(sv, 9) + dv, -1)
                for t in range(16):
                    pk = pkv[t]
                    comp_p[cnt3] = pk
                    cnt3 = jnp.where(pk >= 0, cnt3 + 1, cnt3)
                return cnt3

            cnt2 = lax.fori_loop(0, _SCANC // 16, s16, cnt2)
            nb = cnt2 // _K
            build_flush(nb, cnt2, False)

            def mv(t, cr):
                comp_p[t] = comp_p[nb * _K + t]
                return cr

            lax.fori_loop(0, _K, mv, 0)
            return cnt2 - nb * _K

        return lax.fori_loop(0, _CHUNK_E // _SCANC, sub, cnt)

    cnt = lax.fori_loop(0, _TILE_EDGES // _CHUNK_E, chunk_body, jnp.int32(0))
    build_flush((cnt + _K - 1) // _K, cnt, True)


@functools.partial(
    pl.kernel,
    mesh=_mesh(),
    out_type=jax.ShapeDtypeStruct((2 * _ACC_DST_ROWS, _C), jnp.float32),
    scratch_types=[
        pltpu.VMEM((_CHUNK_E,), jnp.int32),
        pltpu.VMEM((_CHUNK_E,), jnp.int32),
        pltpu.VMEM((_K,), jnp.int32),
        pltpu.VMEM((_K,), jnp.int32),
        pltpu.VMEM((_K,), jnp.int32),
        pltpu.VMEM((_K,), jnp.int32),
        pltpu.VMEM((_K, _C), jnp.float32),
        pltpu.VMEM((_K, _C), jnp.float32),
        pltpu.VMEM((32, _C), jnp.float32),
        pltpu.VMEM_SHARED((_ACC_DST_ROWS, _C), jnp.float32),
        pltpu.SemaphoreType.DMA,
        pltpu.SemaphoreType.DMA,
    ],
)
def _segsum_dst_kernel(hsrc, sd0, sd1, mdst_out,
                       idx0_v, idx1_v, ir0, iw0, ir1, iw1,
                       rows0, rows1, zbuf, acc_dst, sem0, sem1):
    c = lax.axis_index("c")
    s = lax.axis_index("s")
    wid = s * _NC + c
    _zero_rows(zbuf, 32)
    for i in range(20):
        pltpu.sync_copy(zbuf, acc_dst.at[pl.ds(s * 640 + i * 32, 32)])
    plsc.subcore_barrier()
    _db_pass(hsrc, sd0, sd1, acc_dst, idx0_v, idx1_v, ir0, iw0, ir1, iw1,
             rows0, rows1, sem0, sem1, wid, None)
    plsc.subcore_barrier()
    for i in range(20):
        r0 = s * 640 + i * 32
        pltpu.sync_copy(acc_dst.at[pl.ds(r0, 32)], zbuf)
        pltpu.sync_copy(zbuf, mdst_out.at[pl.ds(c * _ACC_DST_ROWS + r0, 32)])


@functools.partial(
    pl.kernel,
    mesh=_mesh(),
    out_type=jax.ShapeDtypeStruct((2 * _SEEDS, _C), jnp.float32),
    scratch_types=[
        pltpu.VMEM((_CHUNK_E,), jnp.int32),
        pltpu.VMEM((_CHUNK_E,), jnp.int32),
        pltpu.VMEM((_K,), jnp.int32),
        pltpu.VMEM((_K,), jnp.int32),
        pltpu.VMEM((_K, _C), jnp.float32),
        pltpu.VMEM((40, _C), jnp.float32),
        pltpu.SMEM((_COMP_CAP,), jnp.int32),
        pltpu.VMEM_SHARED((_ACC_SRC_ROWS, _C), jnp.float32),
        pltpu.SemaphoreType.DMA,
    ],
)
def _segsum1_kernel(hdst2, ds0, ds1, msrc2_out, idx0_v, idx1_v, ir0, iw0,
                    rows0, zbuf, comp_p, acc_src, sem0):
    c = lax.axis_index("c")
    s = lax.axis_index("s")
    wid = s * _NC + c
    _zero_rows(zbuf, 40)
    pltpu.sync_copy(zbuf, acc_src.at[pl.ds(s * 40, 40)])
    plsc.subcore_barrier()
    _filtered_pass(hdst2, ds0, ds1, acc_src, idx0_v, idx1_v, ir0, iw0,
                   comp_p, rows0, sem0, wid)
    plsc.subcore_barrier()
    pltpu.sync_copy(acc_src.at[pl.ds(s * 32, 32)], zbuf.at[pl.ds(0, 32)])
    pltpu.sync_copy(zbuf.at[pl.ds(0, 32)],
                    msrc2_out.at[pl.ds(c * _SEEDS + s * 32, 32)])


def _enc_body(xs_ref, xd_ref, batch_ref, nt_ref, st_ref, fr_ref, Wes_ref,
              bes_ref, Wed_ref, bed_ref, Wt_ref, bt_ref, hs_ref, hd_ref):
    f32 = jnp.float32
    hs = jnp.dot(xs_ref[...], Wes_ref[...], preferred_element_type=f32)
    hs = hs + bes_ref[...]
    batch = batch_ref[...][0]
    oh = batch == lax.broadcasted_iota(jnp.int32, (_BR, _SEEDS), 1)
    seedg = jnp.sum(jnp.where(oh, st_ref[...], 0.0), axis=1, keepdims=True)
    rel = seedg - nt_ref[...][0]
    ang = rel * fr_ref[...]
    pe = jnp.concatenate([jnp.sin(ang), jnp.cos(ang)], axis=1)
    hs = hs + jnp.dot(pe, Wt_ref[...], preferred_element_type=f32)
    hs_ref[...] = hs + bt_ref[...]
    hd = jnp.dot(xd_ref[...], Wed_ref[...], preferred_element_type=f32)
    hd_ref[...] = hd + bed_ref[...]


def _encoder(x_src, x_dst, batch_col, nt_col, seed_row, freqs,
             Wes, bes, Wed, bed, Wt, bt):
    nb = _N // _BR
    row = pl.BlockSpec((_BR, _C), lambda i: (i, 0))
    col = pl.BlockSpec((1, _BR, 1), lambda i: (i, 0, 0))
    return pl.pallas_call(
        _enc_body,
        grid=(nb,),
        in_specs=[
            row,
            row,
            col,
            col,
            pl.BlockSpec((1, _SEEDS), lambda i: (0, 0)),
            pl.BlockSpec((1, _NF), lambda i: (0, 0)),
            pl.BlockSpec((_C, _C), lambda i: (0, 0)),
            pl.BlockSpec((1, _C), lambda i: (0, 0)),
            pl.BlockSpec((_C, _C), lambda i: (0, 0)),
            pl.BlockSpec((1, _C), lambda i: (0, 0)),
            pl.BlockSpec((2 * _NF, _C), lambda i: (0, 0)),
            pl.BlockSpec((1, _C), lambda i: (0, 0)),
        ],
        out_specs=[row, row],
        out_shape=[
            jax.ShapeDtypeStruct((_N, _C), jnp.float32),
            jax.ShapeDtypeStruct((_N, _C), jnp.float32),
        ],
    )(x_src, x_dst, batch_col, nt_col, seed_row, freqs,
      Wes, bes, Wed, bed, Wt, bt)


def _l0_body(hd_ref, m0_ref, m1_ref, Wsd_ref, Wn_ref, out_ref):
    f32 = jnp.float32
    m = m0_ref[...][0] + m1_ref[...][0]
    x = jnp.dot(hd_ref[...], Wsd_ref[...], preferred_element_type=f32)
    x = x + jnp.dot(m, Wn_ref[...], preferred_element_type=f32)
    out_ref[...] = jnp.maximum(x, 0.0)


def _layer0_dst(hd, mdst3, Wsd, Wn):
    nb = _N // _BR
    row = pl.BlockSpec((_BR, _C), lambda i: (i, 0))
    sq = pl.BlockSpec((_C, _C), lambda i: (0, 0))
    return pl.pallas_call(
        _l0_body,
        grid=(nb,),
        in_specs=[
            row,
            pl.BlockSpec((1, _BR, _C), lambda i: (0, i, 0)),
            pl.BlockSpec((1, _BR, _C), lambda i: (1, i, 0)),
            sq,
            sq,
        ],
        out_specs=row,
        out_shape=jax.ShapeDtypeStruct((_N, _C), jnp.float32),
    )(hd, mdst3, mdst3, Wsd, Wn)


def _head_body(hs_ref, m0_ref, m1_ref, n0_ref, n1_ref, Wss0_ref, Wnd0_ref,
               Wss1_ref, Wnd1_ref, Wh_ref, bh_ref, out_ref):
    f32 = jnp.float32
    m = m0_ref[...] + m1_ref[...]
    h2 = jnp.dot(hs_ref[...], Wss0_ref[...], preferred_element_type=f32)
    h2 = jnp.maximum(h2 + jnp.dot(m, Wnd0_ref[...],
                                  preferred_element_type=f32), 0.0)
    n = n0_ref[...] + n1_ref[...]
    h3 = jnp.dot(h2, Wss1_ref[...], preferred_element_type=f32)
    h3 = h3 + jnp.dot(n, Wnd1_ref[...], preferred_element_type=f32)
    out_ref[...] = jnp.dot(h3, Wh_ref[...],
                           preferred_element_type=f32) + bh_ref[...]


def _head(hs, msrc_flat, msrc2_flat, Wss0, Wnd0, Wss1, Wnd1, Wh, bh):
    top = pl.BlockSpec((_SEEDS, _C), lambda i: (0, 0))
    bot = pl.BlockSpec((_SEEDS, _C), lambda i: (1, 0))
    sq = pl.BlockSpec((_C, _C), lambda i: (0, 0))
    return pl.pallas_call(
        _head_body,
        grid=(1,),
        in_specs=[
            top, top, bot, top, bot, sq, sq, sq, sq,
            pl.BlockSpec((_C, _NUM_DST), lambda i: (0, 0)),
            pl.BlockSpec((1, _NUM_DST), lambda i: (0, 0)),
        ],
        out_specs=pl.BlockSpec((_SEEDS, _NUM_DST), lambda i: (0, 0)),
        out_shape=jax.ShapeDtypeStruct((_SEEDS, _NUM_DST), jnp.float32),
    )(hs, msrc_flat, msrc_flat, msrc2_flat, msrc2_flat,
      Wss0, Wnd0, Wss1, Wnd1, Wh, bh)


def kernel(x_src, x_dst, seed_time, node_time_src, W_enc_src, b_enc_src,
           W_enc_dst, b_enc_dst, W_time, b_time, W_self_src_0, W_neigh_ds_0,
           W_self_dst_0, W_neigh_sd_0, W_self_src_1, W_neigh_ds_1,
           W_self_dst_1, W_neigh_sd_1, W_head, b_head, edge_index_sd,
           edge_index_ds, batch_src):
    f32 = jnp.float32
    sd = edge_index_sd.astype(jnp.int32)
    ds = edge_index_ds.astype(jnp.int32)
    sd0, sd1 = sd[0], sd[1]
    ds0, ds1 = ds[0], ds[1]
    batch_col = batch_src.astype(jnp.int32).reshape(_N // _BR, _BR, 1)
    nt_col = node_time_src.reshape(_N // _BR, _BR, 1)
    seed_row = seed_time.reshape(1, _SEEDS)
    freqs = (2.0 ** jnp.arange(_NF, dtype=f32)).reshape(1, _NF)

    hs, hd = _encoder(x_src, x_dst, batch_col, nt_col, seed_row, freqs,
                      W_enc_src, b_enc_src.reshape(1, _C),
                      W_enc_dst, b_enc_dst.reshape(1, _C),
                      W_time, b_time.reshape(1, _C))

    mdst_flat = _segsum_dst_kernel(hs, sd0, sd1)
    msrc_flat = _segsum1_kernel(hd, ds0, ds1)
    mdst3 = mdst_flat.reshape(2, _ACC_DST_ROWS, _C)
    hd2 = _layer0_dst(hd, mdst3, W_self_dst_0, W_neigh_sd_0)
    msrc2_flat = _segsum1_kernel(hd2, ds0, ds1)
    out = _head(hs, msrc_flat, msrc2_flat, W_self_src_0, W_neigh_ds_0,
                W_self_src_1, W_neigh_ds_1, W_head,
                b_head.reshape(1, _NUM_DST))
    return out
